# baseline (speedup 1.0000x reference)
import functools

import jax
import jax.numpy as jnp
from jax.experimental import pallas as pl
from jax.experimental.pallas import tpu as pltpu

BS = 16384
L = 32
NC = 10
EMBD = 256
ZD = 128
K = 128
T = 8
D = NC

CHUNK = 1024
NSTEPS = BS // CHUNK

_ROWC = (((0,), (0,)), ((), ()))
_TRB = (((1,), (1,)), ((), ()))


def _eye(n):
    r = jax.lax.broadcasted_iota(jnp.int32, (n, n), 0)
    c = jax.lax.broadcasted_iota(jnp.int32, (n, n), 1)
    return (r == c).astype(jnp.float32)


def _body(pr_ref, valid_ref, wcnn_ref, bcnn_ref, cb_ref, c2_ref, wfc_ref,
          bfc_ref, wmu_ref, bmu_ref, out_ref, cmt_ref, perp_ref,
          wbig_s, bcnn_s, mbig_s, ctall_s, bout_s, counts_s, acc_s):
    i = pl.program_id(0)

    @pl.when(i == 0)
    def _prep():
        counts_s[...] = jnp.zeros_like(counts_s)
        acc_s[...] = jnp.zeros_like(acc_s)
        wc = jax.lax.dot_general(_eye(4), wcnn_ref[...], _TRB,
                                 preferred_element_type=jnp.float32)
        wbig_s[...] = jnp.zeros_like(wbig_s)
        cbT = jax.lax.dot_general(_eye(NC), cb_ref[...], _TRB,
                                  preferred_element_type=jnp.float32)
        mbig_s[...] = jnp.zeros_like(mbig_s)
        r = jax.lax.broadcasted_iota(jnp.int32, (NC * T, NC * T), 0)
        c = jax.lax.broadcasted_iota(jnp.int32, (NC * T, NC * T), 1)
        pi = (c == (r % NC) * T + r // NC).astype(jnp.float32)
        a_fold = jax.lax.dot_general(pi, wfc_ref[...], _TRB,
                                     preferred_element_type=jnp.float32)
        cb = cb_ref[...]
        for t in range(T):
            wbig_s[4 * t:4 * t + 4, NC * t:NC * t + NC] = wc
            bcnn_s[:, NC * t:NC * t + NC] = bcnn_ref[...]
            mbig_s[NC * t:NC * t + NC, K * t:K * t + K] = -2.0 * cbT
            m1 = jnp.dot(cb, a_fold[NC * t:NC * t + NC, :],
                         preferred_element_type=jnp.float32)
            ctall_s[K * t:K * t + K, :] = jax.lax.dot_general(
                m1, wmu_ref[...], _TRB,
                preferred_element_type=jnp.float32).astype(jnp.bfloat16)
        bout_s[...] = jax.lax.dot_general(
            bfc_ref[...], wmu_ref[...], _TRB,
            preferred_element_type=jnp.float32) + bmu_ref[...]

    pr = pr_ref[...]
    valid = valid_ref[...]

    z_all = jnp.dot(pr, wbig_s[...], preferred_element_type=jnp.float32)
    z_all = jnp.maximum(z_all + bcnn_s[...], 0.0)

    s_all = jnp.dot(z_all, mbig_s[...],
                    preferred_element_type=jnp.float32) + c2_ref[...]

    iota = jax.lax.broadcasted_iota(jnp.int32, (CHUNK, K), 1)
    encs = []
    us = []
    for t in range(T):
        s_t = s_all[:, t * K:(t + 1) * K]
        amin = jnp.argmin(s_t, axis=1).astype(jnp.int32)
        msk = iota == amin[:, None]
        encs.append(msk.astype(jnp.bfloat16))
        us.append(jnp.where(msk, s_t, 0.0))
    enc_all = jnp.concatenate(encs, axis=1)
    u_all = jnp.concatenate(us, axis=1)

    out_ref[...] = jnp.dot(
        enc_all, ctall_s[...],
        preferred_element_type=jnp.float32) + bout_s[...]

    validb = valid.astype(jnp.bfloat16)
    counts_s[...] = counts_s[...] + jax.lax.dot_general(
        validb, enc_all, _ROWC, preferred_element_type=jnp.float32)
    z2sum = jnp.sum(z_all * z_all, axis=1, keepdims=True)
    lossvec = jax.lax.dot_general(valid, u_all, _ROWC,
                                  preferred_element_type=jnp.float32)
    loss = (jnp.sum(lossvec, axis=1, keepdims=True)
            + jax.lax.dot_general(valid, z2sum, _ROWC,
                                  preferred_element_type=jnp.float32))
    vsum = jnp.sum(valid).reshape(1, 1)
    acc_s[...] = acc_s[...] + jnp.concatenate([loss, vsum], axis=1)

    @pl.when(i == NSTEPS - 1)
    def _fin():
        a = acc_s[...]
        loss_sum = a[:, 0:1]
        n8 = a[:, 1:2] * T
        e_latent = loss_sum / (n8 * D + 1e-9)
        cmt_ref[...] = 0.25 * e_latent
        call = counts_s[...]
        c128 = call[:, 0:K]
        for t in range(1, T):
            c128 = c128 + call[:, t * K:(t + 1) * K]
        p = c128 / (n8 + 1e-9)
        ent = -jnp.sum(p * jnp.log(p + 1e-10), axis=1, keepdims=True)
        perp_ref[...] = jnp.exp(ent)


@functools.partial(jax.jit, static_argnames=())
def kernel(pr, track_pad_mask, W_cnn, b_cnn, codebook, W_fc, b_fc, W_mu, b_mu):
    validf = 1.0 - track_pad_mask.astype(jnp.float32)
    c2t = jnp.tile(jnp.sum(codebook * codebook, axis=1), T)[None, :]

    out, cmt, perp = pl.pallas_call(
        _body,
        grid=(NSTEPS,),
        in_specs=[
            pl.BlockSpec((CHUNK, L), lambda i: (i, 0)),
            pl.BlockSpec((CHUNK, 1), lambda i: (i, 0)),
            pl.BlockSpec((NC, 4), lambda i: (0, 0)),
            pl.BlockSpec((1, NC), lambda i: (0, 0)),
            pl.BlockSpec((K, NC), lambda i: (0, 0)),
            pl.BlockSpec((1, T * K), lambda i: (0, 0)),
            pl.BlockSpec((EMBD, NC * T), lambda i: (0, 0)),
            pl.BlockSpec((1, EMBD), lambda i: (0, 0)),
            pl.BlockSpec((ZD, EMBD), lambda i: (0, 0)),
            pl.BlockSpec((1, ZD), lambda i: (0, 0)),
        ],
        out_specs=[
            pl.BlockSpec((CHUNK, ZD), lambda i: (i, 0)),
            pl.BlockSpec((1, 1), lambda i: (0, 0)),
            pl.BlockSpec((1, 1), lambda i: (0, 0)),
        ],
        out_shape=[
            jax.ShapeDtypeStruct((BS, ZD), jnp.float32),
            jax.ShapeDtypeStruct((1, 1), jnp.float32),
            jax.ShapeDtypeStruct((1, 1), jnp.float32),
        ],
        scratch_shapes=[
            pltpu.VMEM((L, NC * T), jnp.float32),
            pltpu.VMEM((1, NC * T), jnp.float32),
            pltpu.VMEM((NC * T, T * K), jnp.float32),
            pltpu.VMEM((T * K, ZD), jnp.bfloat16),
            pltpu.VMEM((1, ZD), jnp.float32),
            pltpu.VMEM((1, T * K), jnp.float32),
            pltpu.VMEM((1, 2), jnp.float32),
        ],
    )(pr, validf, W_cnn.reshape(NC, 4), b_cnn[None, :], codebook, c2t,
      W_fc, b_fc[None, :], W_mu, b_mu[None, :])

    return (out, cmt[0, 0], perp[0, 0])


# speedup vs baseline: 1.0728x; 1.0728x over previous
import functools

import jax
import jax.numpy as jnp
from jax.experimental import pallas as pl
from jax.experimental.pallas import tpu as pltpu

BS = 16384
L = 32
NC = 10
EMBD = 256
ZD = 128
K = 128
T = 8
D = NC

CHUNK = 4096
NSTEPS = BS // CHUNK

_ROWC = (((0,), (0,)), ((), ()))
_TRB = (((1,), (1,)), ((), ()))


def _eye(n):
    r = jax.lax.broadcasted_iota(jnp.int32, (n, n), 0)
    c = jax.lax.broadcasted_iota(jnp.int32, (n, n), 1)
    return (r == c).astype(jnp.float32)


def _body(pr_ref, valid_ref, wcnn_ref, bcnn_ref, cb_ref, c2_ref, wfc_ref,
          bfc_ref, wmu_ref, bmu_ref, out_ref, cmt_ref, perp_ref,
          wbig_s, bcnn_s, mbig_s, ctall_s, bout_s, counts_s, acc_s):
    i = pl.program_id(0)

    @pl.when(i == 0)
    def _prep():
        counts_s[...] = jnp.zeros_like(counts_s)
        acc_s[...] = jnp.zeros_like(acc_s)
        wc = jax.lax.dot_general(_eye(4), wcnn_ref[...], _TRB,
                                 preferred_element_type=jnp.float32)
        wbig_s[...] = jnp.zeros_like(wbig_s)
        cbT = jax.lax.dot_general(_eye(NC), cb_ref[...], _TRB,
                                  preferred_element_type=jnp.float32)
        mbig_s[...] = jnp.zeros_like(mbig_s)
        r = jax.lax.broadcasted_iota(jnp.int32, (NC * T, NC * T), 0)
        c = jax.lax.broadcasted_iota(jnp.int32, (NC * T, NC * T), 1)
        pi = (c == (r % NC) * T + r // NC).astype(jnp.float32)
        a_fold = jax.lax.dot_general(pi, wfc_ref[...], _TRB,
                                     preferred_element_type=jnp.float32)
        cb = cb_ref[...]
        for t in range(T):
            wbig_s[4 * t:4 * t + 4, NC * t:NC * t + NC] = wc
            bcnn_s[:, NC * t:NC * t + NC] = bcnn_ref[...]
            mbig_s[NC * t:NC * t + NC, K * t:K * t + K] = -2.0 * cbT
            m1 = jnp.dot(cb, a_fold[NC * t:NC * t + NC, :],
                         preferred_element_type=jnp.float32)
            ctall_s[K * t:K * t + K, :] = jax.lax.dot_general(
                m1, wmu_ref[...], _TRB,
                preferred_element_type=jnp.float32).astype(jnp.bfloat16)
        bout_s[...] = jax.lax.dot_general(
            bfc_ref[...], wmu_ref[...], _TRB,
            preferred_element_type=jnp.float32) + bmu_ref[...]

    pr = pr_ref[...]
    valid = valid_ref[...]

    z_all = jnp.dot(pr, wbig_s[...], preferred_element_type=jnp.float32)
    z_all = jnp.maximum(z_all + bcnn_s[...], 0.0)

    s_all = jnp.dot(z_all, mbig_s[...],
                    preferred_element_type=jnp.float32) + c2_ref[...]

    iota = jax.lax.broadcasted_iota(jnp.int32, (CHUNK, K), 1)
    encs = []
    us = []
    for t in range(T):
        s_t = s_all[:, t * K:(t + 1) * K]
        amin = jnp.argmin(s_t, axis=1).astype(jnp.int32)
        msk = iota == amin[:, None]
        encs.append(msk.astype(jnp.bfloat16))
        us.append(jnp.where(msk, s_t, 0.0))
    enc_all = jnp.concatenate(encs, axis=1)
    u_all = jnp.concatenate(us, axis=1)

    out_ref[...] = jnp.dot(
        enc_all, ctall_s[...],
        preferred_element_type=jnp.float32) + bout_s[...]

    validb = valid.astype(jnp.bfloat16)
    counts_s[...] = counts_s[...] + jax.lax.dot_general(
        validb, enc_all, _ROWC, preferred_element_type=jnp.float32)
    z2sum = jnp.sum(z_all * z_all, axis=1, keepdims=True)
    lossvec = jax.lax.dot_general(valid, u_all, _ROWC,
                                  preferred_element_type=jnp.float32)
    loss = (jnp.sum(lossvec, axis=1, keepdims=True)
            + jax.lax.dot_general(valid, z2sum, _ROWC,
                                  preferred_element_type=jnp.float32))
    vsum = jnp.sum(valid).reshape(1, 1)
    acc_s[...] = acc_s[...] + jnp.concatenate([loss, vsum], axis=1)

    @pl.when(i == NSTEPS - 1)
    def _fin():
        a = acc_s[...]
        loss_sum = a[:, 0:1]
        n8 = a[:, 1:2] * T
        e_latent = loss_sum / (n8 * D + 1e-9)
        cmt_ref[...] = 0.25 * e_latent
        call = counts_s[...]
        c128 = call[:, 0:K]
        for t in range(1, T):
            c128 = c128 + call[:, t * K:(t + 1) * K]
        p = c128 / (n8 + 1e-9)
        ent = -jnp.sum(p * jnp.log(p + 1e-10), axis=1, keepdims=True)
        perp_ref[...] = jnp.exp(ent)


@functools.partial(jax.jit, static_argnames=())
def kernel(pr, track_pad_mask, W_cnn, b_cnn, codebook, W_fc, b_fc, W_mu, b_mu):
    validf = 1.0 - track_pad_mask.astype(jnp.float32)
    c2t = jnp.tile(jnp.sum(codebook * codebook, axis=1), T)[None, :]

    out, cmt, perp = pl.pallas_call(
        _body,
        grid=(NSTEPS,),
        in_specs=[
            pl.BlockSpec((CHUNK, L), lambda i: (i, 0)),
            pl.BlockSpec((CHUNK, 1), lambda i: (i, 0)),
            pl.BlockSpec((NC, 4), lambda i: (0, 0)),
            pl.BlockSpec((1, NC), lambda i: (0, 0)),
            pl.BlockSpec((K, NC), lambda i: (0, 0)),
            pl.BlockSpec((1, T * K), lambda i: (0, 0)),
            pl.BlockSpec((EMBD, NC * T), lambda i: (0, 0)),
            pl.BlockSpec((1, EMBD), lambda i: (0, 0)),
            pl.BlockSpec((ZD, EMBD), lambda i: (0, 0)),
            pl.BlockSpec((1, ZD), lambda i: (0, 0)),
        ],
        out_specs=[
            pl.BlockSpec((CHUNK, ZD), lambda i: (i, 0)),
            pl.BlockSpec((1, 1), lambda i: (0, 0)),
            pl.BlockSpec((1, 1), lambda i: (0, 0)),
        ],
        out_shape=[
            jax.ShapeDtypeStruct((BS, ZD), jnp.float32),
            jax.ShapeDtypeStruct((1, 1), jnp.float32),
            jax.ShapeDtypeStruct((1, 1), jnp.float32),
        ],
        scratch_shapes=[
            pltpu.VMEM((L, NC * T), jnp.float32),
            pltpu.VMEM((1, NC * T), jnp.float32),
            pltpu.VMEM((NC * T, T * K), jnp.float32),
            pltpu.VMEM((T * K, ZD), jnp.bfloat16),
            pltpu.VMEM((1, ZD), jnp.float32),
            pltpu.VMEM((1, T * K), jnp.float32),
            pltpu.VMEM((1, 2), jnp.float32),
        ],
    )(pr, validf, W_cnn.reshape(NC, 4), b_cnn[None, :], codebook, c2t,
      W_fc, b_fc[None, :], W_mu, b_mu[None, :])

    return (out, cmt[0, 0], perp[0, 0])


# speedup vs baseline: 1.2231x; 1.1401x over previous
import functools

import jax
import jax.numpy as jnp
from jax.experimental import pallas as pl
from jax.experimental.pallas import tpu as pltpu

BS = 16384
L = 32
NC = 10
EMBD = 256
ZD = 128
K = 128
T = 8
D = NC

CHUNK = 4096
NSTEPS = BS // CHUNK

_ROWC = (((0,), (0,)), ((), ()))
_TRB = (((1,), (1,)), ((), ()))


def _eye(n):
    r = jax.lax.broadcasted_iota(jnp.int32, (n, n), 0)
    c = jax.lax.broadcasted_iota(jnp.int32, (n, n), 1)
    return (r == c).astype(jnp.float32)


def _body(pr_ref, mask_ref, wcnn_ref, bcnn_ref, cb_ref, wfc_ref,
          bfc_ref, wmu_ref, bmu_ref, out_ref, cmt_ref, perp_ref,
          wbig_s, bcnn_s, mbig_s, ctall_s, bout_s, c2_s, counts_s, acc_s):
    i = pl.program_id(0)

    @pl.when(i == 0)
    def _prep():
        counts_s[...] = jnp.zeros_like(counts_s)
        acc_s[...] = jnp.zeros_like(acc_s)
        wc = jax.lax.dot_general(_eye(4), wcnn_ref[...], _TRB,
                                 preferred_element_type=jnp.float32)
        wbig_s[...] = jnp.zeros_like(wbig_s)
        cbT = jax.lax.dot_general(_eye(NC), cb_ref[...], _TRB,
                                  preferred_element_type=jnp.float32)
        mbig_s[...] = jnp.zeros_like(mbig_s)
        r = jax.lax.broadcasted_iota(jnp.int32, (NC * T, NC * T), 0)
        c = jax.lax.broadcasted_iota(jnp.int32, (NC * T, NC * T), 1)
        pi = (c == (r % NC) * T + r // NC).astype(jnp.float32)
        a_fold = jax.lax.dot_general(pi, wfc_ref[...], _TRB,
                                     preferred_element_type=jnp.float32)
        cb = cb_ref[...]
        c2col = jnp.sum(cb * cb, axis=1, keepdims=True)
        c2row = jnp.transpose(c2col, (1, 0))
        for t in range(T):
            c2_s[:, K * t:K * t + K] = c2row
            wbig_s[4 * t:4 * t + 4, NC * t:NC * t + NC] = wc
            bcnn_s[:, NC * t:NC * t + NC] = bcnn_ref[...]
            mbig_s[NC * t:NC * t + NC, K * t:K * t + K] = -2.0 * cbT
            m1 = jnp.dot(cb, a_fold[NC * t:NC * t + NC, :],
                         preferred_element_type=jnp.float32)
            ctall_s[K * t:K * t + K, :] = jax.lax.dot_general(
                m1, wmu_ref[...], _TRB,
                preferred_element_type=jnp.float32).astype(jnp.bfloat16)
        bout_s[...] = jax.lax.dot_general(
            bfc_ref[...], wmu_ref[...], _TRB,
            preferred_element_type=jnp.float32) + bmu_ref[...]

    pr = pr_ref[...]
    valid = 1.0 - mask_ref[...].astype(jnp.float32)

    z_all = jnp.dot(pr, wbig_s[...], preferred_element_type=jnp.float32)
    z_all = jnp.maximum(z_all + bcnn_s[...], 0.0)

    s_all = jnp.dot(z_all, mbig_s[...],
                    preferred_element_type=jnp.float32) + c2_s[...]

    iota = jax.lax.broadcasted_iota(jnp.int32, (CHUNK, K), 1)
    encs = []
    us = []
    for t in range(T):
        s_t = s_all[:, t * K:(t + 1) * K]
        amin = jnp.argmin(s_t, axis=1).astype(jnp.int32)
        msk = iota == amin[:, None]
        encs.append(msk.astype(jnp.bfloat16))
        us.append(jnp.where(msk, s_t, 0.0))
    enc_all = jnp.concatenate(encs, axis=1)
    u_all = jnp.concatenate(us, axis=1)

    out_ref[...] = jnp.dot(
        enc_all, ctall_s[...],
        preferred_element_type=jnp.float32) + bout_s[...]

    validb = valid.astype(jnp.bfloat16)
    counts_s[...] = counts_s[...] + jax.lax.dot_general(
        validb, enc_all, _ROWC, preferred_element_type=jnp.float32)
    z2sum = jnp.sum(z_all * z_all, axis=1, keepdims=True)
    lossvec = jax.lax.dot_general(valid, u_all, _ROWC,
                                  preferred_element_type=jnp.float32)
    loss = (jnp.sum(lossvec, axis=1, keepdims=True)
            + jax.lax.dot_general(valid, z2sum, _ROWC,
                                  preferred_element_type=jnp.float32))
    vsum = jnp.sum(valid).reshape(1, 1)
    acc_s[...] = acc_s[...] + jnp.concatenate([loss, vsum], axis=1)

    @pl.when(i == NSTEPS - 1)
    def _fin():
        a = acc_s[...]
        loss_sum = a[:, 0:1]
        n8 = a[:, 1:2] * T
        e_latent = loss_sum / (n8 * D + 1e-9)
        cmt_ref[...] = 0.25 * e_latent
        call = counts_s[...]
        c128 = call[:, 0:K]
        for t in range(1, T):
            c128 = c128 + call[:, t * K:(t + 1) * K]
        p = c128 / (n8 + 1e-9)
        ent = -jnp.sum(p * jnp.log(p + 1e-10), axis=1, keepdims=True)
        perp_ref[...] = jnp.exp(ent)


@functools.partial(jax.jit, static_argnames=())
def kernel(pr, track_pad_mask, W_cnn, b_cnn, codebook, W_fc, b_fc, W_mu, b_mu):

    out, cmt, perp = pl.pallas_call(
        _body,
        grid=(NSTEPS,),
        in_specs=[
            pl.BlockSpec((CHUNK, L), lambda i: (i, 0)),
            pl.BlockSpec((CHUNK, 1), lambda i: (i, 0)),
            pl.BlockSpec((NC, 4), lambda i: (0, 0)),
            pl.BlockSpec((1, NC), lambda i: (0, 0)),
            pl.BlockSpec((K, NC), lambda i: (0, 0)),
            pl.BlockSpec((EMBD, NC * T), lambda i: (0, 0)),
            pl.BlockSpec((1, EMBD), lambda i: (0, 0)),
            pl.BlockSpec((ZD, EMBD), lambda i: (0, 0)),
            pl.BlockSpec((1, ZD), lambda i: (0, 0)),
        ],
        out_specs=[
            pl.BlockSpec((CHUNK, ZD), lambda i: (i, 0)),
            pl.BlockSpec((1, 1), lambda i: (0, 0)),
            pl.BlockSpec((1, 1), lambda i: (0, 0)),
        ],
        out_shape=[
            jax.ShapeDtypeStruct((BS, ZD), jnp.float32),
            jax.ShapeDtypeStruct((1, 1), jnp.float32),
            jax.ShapeDtypeStruct((1, 1), jnp.float32),
        ],
        scratch_shapes=[
            pltpu.VMEM((L, NC * T), jnp.float32),
            pltpu.VMEM((1, NC * T), jnp.float32),
            pltpu.VMEM((NC * T, T * K), jnp.float32),
            pltpu.VMEM((T * K, ZD), jnp.bfloat16),
            pltpu.VMEM((1, ZD), jnp.float32),
            pltpu.VMEM((1, T * K), jnp.float32),
            pltpu.VMEM((1, T * K), jnp.float32),
            pltpu.VMEM((1, 2), jnp.float32),
        ],
    )(pr, track_pad_mask, W_cnn.reshape(NC, 4), b_cnn[None, :], codebook,
      W_fc, b_fc[None, :], W_mu, b_mu[None, :])

    return (out, cmt[0, 0], perp[0, 0])
